# baseline (speedup 1.0000x reference)
import functools

import jax
import jax.numpy as jnp
from jax import lax
from jax.experimental import pallas as pl
from jax.experimental.pallas import tpu as pltpu
from jax.experimental.pallas import tpu_sc as plsc

_SIZE = 1000000
_DIM = 64
_BATCH = 16384
_FIELDS = 26

_NC = 2
_NS = 16
_NW = _NC * _NS
_TOTAL = _BATCH * _FIELDS
_PER_W = _TOTAL // _NW
_C = 128
_NCH = _PER_W // _C


def _make_gather():
    mesh = plsc.VectorSubcoreMesh(core_axis_name="c", subcore_axis_name="s")

    @functools.partial(
        pl.kernel,
        mesh=mesh,
        compiler_params=pltpu.CompilerParams(use_tc_tiling_on_sc=False),
        out_type=jax.ShapeDtypeStruct((_TOTAL, _DIM), jnp.float32),
        scratch_types=[
            pltpu.VMEM((_NCH, _C), jnp.int32),
            pltpu.VMEM((_C, _DIM), jnp.float32),
            pltpu.SemaphoreType.DMA,
        ],
    )
    def gather_k(table_hbm, idx_hbm, out_hbm, idx_v, rows_v, sem):
        wid = lax.axis_index("s") * _NC + lax.axis_index("c")
        pltpu.sync_copy(idx_hbm.at[wid], idx_v)
        base = wid * _PER_W

        def body(c, carry):
            pltpu.async_copy(table_hbm.at[idx_v.at[c]], rows_v, sem).wait()
            pltpu.sync_copy(rows_v, out_hbm.at[pl.ds(base + c * _C, _C)])
            return carry

        lax.fori_loop(0, _NCH, body, 0)

    return gather_k


_gather = _make_gather()


@jax.jit
def kernel(batch, embeddings):
    idx = batch.astype(jnp.int32).reshape(_NW, _NCH, _C)
    out = _gather(embeddings, idx)
    return out.reshape(_BATCH, _FIELDS, _DIM)


# speedup vs baseline: 1.0772x; 1.0772x over previous
import functools

import jax
import jax.numpy as jnp
from jax import lax
from jax.experimental import pallas as pl
from jax.experimental.pallas import tpu as pltpu
from jax.experimental.pallas import tpu_sc as plsc

_SIZE = 1000000
_DIM = 64
_BATCH = 16384
_FIELDS = 26

_NC = 2
_NS = 16
_NW = _NC * _NS
_TOTAL = _BATCH * _FIELDS
_PER_W = _TOTAL // _NW
_C = 128
_NCH = _PER_W // _C
_GSUB = 4
_G = _C * _GSUB
_NBIG = _PER_W // _G


def _make_gather():
    mesh = plsc.VectorSubcoreMesh(core_axis_name="c", subcore_axis_name="s")

    @functools.partial(
        pl.kernel,
        mesh=mesh,
        compiler_params=pltpu.CompilerParams(use_tc_tiling_on_sc=False),
        out_type=jax.ShapeDtypeStruct((_TOTAL, _DIM), jnp.float32),
        scratch_types=[
            pltpu.VMEM((_NCH, _C), jnp.int32),
            pltpu.VMEM((2, _G, _DIM), jnp.float32),
            pltpu.SemaphoreType.DMA((2,)),
        ],
    )
    def gather_k(table_hbm, idx_hbm, out_hbm, idx_v, rows_v, sems):
        wid = lax.axis_index("s") * _NC + lax.axis_index("c")
        pltpu.sync_copy(idx_hbm.at[wid], idx_v)
        base = wid * _PER_W

        def fire(g, b):
            for j in range(_GSUB):
                pltpu.async_copy(
                    table_hbm.at[idx_v.at[g * _GSUB + j]],
                    rows_v.at[b, pl.ds(j * _C, _C)],
                    sems.at[b],
                )

        fire(0, 0)

        def body(g, carry):
            b = lax.rem(g, 2)
            pltpu.make_async_copy(
                out_hbm.at[pl.ds(0, _G)], rows_v.at[b], sems.at[b]
            ).wait()

            @pl.when(g + 1 < _NBIG)
            def _():
                fire(g + 1, 1 - b)

            pltpu.sync_copy(rows_v.at[b], out_hbm.at[pl.ds(base + g * _G, _G)])
            return carry

        lax.fori_loop(0, _NBIG, body, 0)

    return gather_k


_gather = _make_gather()


@jax.jit
def kernel(batch, embeddings):
    idx = batch.astype(jnp.int32).reshape(_NW, _NCH, _C)
    out = _gather(embeddings, idx)
    return out.reshape(_BATCH, _FIELDS, _DIM)


# speedup vs baseline: 1.0804x; 1.0030x over previous
import functools

import jax
import jax.numpy as jnp
from jax import lax
from jax.experimental import pallas as pl
from jax.experimental.pallas import tpu as pltpu
from jax.experimental.pallas import tpu_sc as plsc

_SIZE = 1000000
_DIM = 64
_BATCH = 16384
_FIELDS = 26

_NC = 2
_NS = 16
_NW = _NC * _NS
_TOTAL = _BATCH * _FIELDS
_PER_W = _TOTAL // _NW
_C = 128
_NCH = _PER_W // _C
_GSUB = 4
_G = _C * _GSUB
_NBIG = _PER_W // _G
_DEPTH = 3


def _make_gather():
    mesh = plsc.VectorSubcoreMesh(core_axis_name="c", subcore_axis_name="s")

    @functools.partial(
        pl.kernel,
        mesh=mesh,
        compiler_params=pltpu.CompilerParams(use_tc_tiling_on_sc=False),
        out_type=jax.ShapeDtypeStruct((_TOTAL, _DIM), jnp.float32),
        scratch_types=[
            pltpu.VMEM((_NCH, _C), jnp.int32),
            pltpu.VMEM((_DEPTH, _G, _DIM), jnp.float32),
            pltpu.SemaphoreType.DMA((_DEPTH,)),
        ],
    )
    def gather_k(table_hbm, idx_hbm, out_hbm, idx_v, rows_v, sems):
        wid = lax.axis_index("s") * _NC + lax.axis_index("c")
        pltpu.sync_copy(idx_hbm.at[wid], idx_v)
        base = wid * _PER_W

        def fire(g, b):
            for j in range(_GSUB):
                pltpu.async_copy(
                    table_hbm.at[idx_v.at[g * _GSUB + j]],
                    rows_v.at[b, pl.ds(j * _C, _C)],
                    sems.at[b],
                )

        for d in range(_DEPTH):
            fire(d, d)

        def body(g, carry):
            b = lax.rem(g, _DEPTH)
            pltpu.make_async_copy(
                out_hbm.at[pl.ds(0, _G)], rows_v.at[b], sems.at[b]
            ).wait()
            pltpu.sync_copy(rows_v.at[b], out_hbm.at[pl.ds(base + g * _G, _G)])

            @pl.when(g + _DEPTH < _NBIG)
            def _():
                fire(g + _DEPTH, b)

            return carry

        lax.fori_loop(0, _NBIG, body, 0)

    return gather_k


_gather = _make_gather()


@jax.jit
def kernel(batch, embeddings):
    idx = batch.astype(jnp.int32).reshape(_NW, _NCH, _C)
    out = _gather(embeddings, idx)
    return out.reshape(_BATCH, _FIELDS, _DIM)
